# baseline (speedup 1.0000x reference)
import functools

import numpy as np
import jax
import jax.numpy as jnp
from jax.experimental import pallas as pl

_H = 1024
_NH = 16
_NKV = 4
_HD = 64
_E = 8
_TOPK = 2
_IM = 512
_ISH = 1024
_THETA = 500000.0
_EPS = 1e-06

_BS = 256
_BQ = 256
_BK = 256


def _rope_half(x, cos, sin):
    x1 = x[:, :32]
    x2 = x[:, 32:]
    return jnp.concatenate([x1 * cos - x2 * sin, x1 * sin + x2 * cos], axis=1)


def _pre_kernel(hid_ref, res_ref, wln_ref, wqkv_ref, res1_ref, qkv_ref):
    h = hid_ref[...] + res_ref[...]
    res1_ref[...] = h
    v = jnp.mean(h * h, axis=1, keepdims=True)
    ln = h * jax.lax.rsqrt(v + _EPS) * wln_ref[...]
    qkv_ref[...] = jnp.dot(ln, wqkv_ref[...], preferred_element_type=jnp.float32)


def _attn_kernel(cosq_ref, sinq_ref, cosk_ref, sink_ref, q_ref, k_ref, v_ref,
                 o_ref):
    i = pl.program_id(1)
    q = _rope_half(q_ref[0], cosq_ref[...], sinq_ref[...]) * (_HD ** -0.5)

    def body(j, carry):
        acc, m, l = carry
        kj = k_ref[0, pl.ds(j * _BK, _BK), :]
        ck = cosk_ref[pl.ds(j * _BK, _BK), :]
        sk = sink_ref[pl.ds(j * _BK, _BK), :]
        kr = _rope_half(kj, ck, sk)
        s = jax.lax.dot_general(q, kr, (((1,), (1,)), ((), ())),
                                preferred_element_type=jnp.float32)
        qpos = i * _BQ + jax.lax.broadcasted_iota(jnp.int32, (_BQ, _BK), 0)
        kpos = j * _BK + jax.lax.broadcasted_iota(jnp.int32, (_BQ, _BK), 1)
        s = jnp.where(qpos >= kpos, s, -1e30)
        m_new = jnp.maximum(m, jnp.max(s, axis=1, keepdims=True))
        alpha = jnp.exp(m - m_new)
        p = jnp.exp(s - m_new)
        vj = v_ref[0, pl.ds(j * _BK, _BK), :]
        l_new = l * alpha + jnp.sum(p, axis=1, keepdims=True)
        acc_new = acc * alpha + jnp.dot(p, vj,
                                        preferred_element_type=jnp.float32)
        return acc_new, m_new, l_new

    acc0 = jnp.zeros((_BQ, _HD), jnp.float32)
    m0 = jnp.full((_BQ, 1), -1e30, jnp.float32)
    l0 = jnp.zeros((_BQ, 1), jnp.float32)
    acc, m, l = jax.lax.fori_loop(0, i + 1, body, (acc0, m0, l0))
    o_ref[0] = acc / l


def _post_kernel(ctx_ref, res1_ref, wln_ref, wo_ref, gwt_ref, gb_ref,
                 res2_ref, h2_ref, we_ref):
    att = jnp.dot(ctx_ref[...], wo_ref[...], preferred_element_type=jnp.float32)
    r2 = att + res1_ref[...]
    res2_ref[...] = r2
    v = jnp.mean(r2 * r2, axis=1, keepdims=True)
    h2 = r2 * jax.lax.rsqrt(v + _EPS) * wln_ref[...]
    h2_ref[...] = h2.astype(jnp.bfloat16)
    logits = jnp.dot(h2, gwt_ref[...], preferred_element_type=jnp.float32)
    mx = jnp.max(logits, axis=1, keepdims=True)
    ex = jnp.exp(logits - mx)
    probs = ex / jnp.sum(ex, axis=1, keepdims=True)
    b = probs + gb_ref[...]
    idx = jax.lax.broadcasted_iota(jnp.int32, (_BS, _E), 1)
    m1 = jnp.max(b, axis=1, keepdims=True)
    a1 = jnp.min(jnp.where(b == m1, idx, _E), axis=1, keepdims=True)
    oh1 = idx == a1
    b2 = jnp.where(oh1, -1e30, b)
    m2 = jnp.max(b2, axis=1, keepdims=True)
    a2 = jnp.min(jnp.where(b2 == m2, idx, _E), axis=1, keepdims=True)
    sel = oh1 | (idx == a2)
    w = jnp.where(sel, probs, 0.0)
    we_ref[...] = w / jnp.sum(w, axis=1, keepdims=True)


def _shared_kernel(h2_ref, gu_ref, dn_ref, out_ref):
    g = jnp.dot(h2_ref[...], gu_ref[...], preferred_element_type=jnp.float32)
    g1 = g[:, :_ISH]
    g2 = g[:, _ISH:]
    a = (g1 * jax.nn.sigmoid(g1) * g2).astype(jnp.bfloat16)
    out_ref[...] = jnp.dot(a, dn_ref[...], preferred_element_type=jnp.float32)


def _moe_kernel(h2_ref, we_ref, shared_ref, gu_ref, dn_ref, out_ref):
    e = pl.program_id(0)
    r = pl.program_id(1)
    g = jnp.dot(h2_ref[...], gu_ref[0], preferred_element_type=jnp.float32)
    g1 = g[:, :_IM]
    g2 = g[:, _IM:]
    a = (g1 * jax.nn.sigmoid(g1) * g2).astype(jnp.bfloat16)
    xe = jnp.dot(a, dn_ref[0], preferred_element_type=jnp.float32)
    idx = jax.lax.broadcasted_iota(jnp.int32, (_BS, _E), 1)
    w = jnp.sum(we_ref[...] * (idx == e).astype(jnp.float32), axis=1,
                keepdims=True)
    contrib = w * xe
    rows = pl.ds(r * _BS, _BS)

    @pl.when(e == 0)
    def _():
        out_ref[rows, :] = shared_ref[...] + contrib

    @pl.when(e != 0)
    def _():
        out_ref[rows, :] = out_ref[rows, :] + contrib


def kernel(hidden_states, residual, w_in_ln, W_qkv, W_o, w_post_ln, gate_w,
           gate_bias, exp_gu, exp_dn, sh_gu, sh_dn, positions):
    T = hidden_states.shape[0]
    nq = _NH * _HD

    half = np.arange(0, _HD, 2)
    head_perm = np.concatenate([half, half + 1])
    qperm = (np.arange(_NH)[:, None] * _HD + head_perm[None, :]).reshape(-1)
    kperm = nq + (np.arange(_NKV)[:, None] * _HD + head_perm[None, :]).reshape(-1)
    vcols = np.arange(nq + _NKV * _HD, nq + 2 * _NKV * _HD)
    perm = np.concatenate([qperm, kperm, vcols])
    wqkv_p = W_qkv[:, perm]

    inv_freq = 1.0 / (_THETA ** (jnp.arange(0, _HD, 2, dtype=jnp.float32) / _HD))
    freqs = positions.astype(jnp.float32)[:, None] * inv_freq[None, :]
    cos_t = jnp.cos(freqs)
    sin_t = jnp.sin(freqs)

    nR = T // _BS

    res1, qkv = pl.pallas_call(
        _pre_kernel,
        grid=(nR,),
        in_specs=[
            pl.BlockSpec((_BS, _H), lambda r: (r, 0)),
            pl.BlockSpec((_BS, _H), lambda r: (r, 0)),
            pl.BlockSpec((1, _H), lambda r: (0, 0)),
            pl.BlockSpec((_H, (_NH + 2 * _NKV) * _HD), lambda r: (0, 0)),
        ],
        out_specs=[
            pl.BlockSpec((_BS, _H), lambda r: (r, 0)),
            pl.BlockSpec((_BS, (_NH + 2 * _NKV) * _HD), lambda r: (r, 0)),
        ],
        out_shape=[
            jax.ShapeDtypeStruct((T, _H), jnp.float32),
            jax.ShapeDtypeStruct((T, (_NH + 2 * _NKV) * _HD), jnp.float32),
        ],
    )(hidden_states, residual, w_in_ln.reshape(1, _H), wqkv_p)

    q = qkv[:, :nq].reshape(T, _NH, _HD).transpose(1, 0, 2)
    k = qkv[:, nq:nq + _NKV * _HD].reshape(T, _NKV, _HD).transpose(1, 0, 2)
    v = qkv[:, nq + _NKV * _HD:].reshape(T, _NKV, _HD).transpose(1, 0, 2)

    rep = _NH // _NKV
    ctx = pl.pallas_call(
        _attn_kernel,
        grid=(_NH, T // _BQ),
        in_specs=[
            pl.BlockSpec((_BQ, _HD // 2), lambda h, i: (i, 0)),
            pl.BlockSpec((_BQ, _HD // 2), lambda h, i: (i, 0)),
            pl.BlockSpec((T, _HD // 2), lambda h, i: (0, 0)),
            pl.BlockSpec((T, _HD // 2), lambda h, i: (0, 0)),
            pl.BlockSpec((1, _BQ, _HD), lambda h, i: (h, i, 0)),
            pl.BlockSpec((1, T, _HD), lambda h, i: (h // rep, 0, 0)),
            pl.BlockSpec((1, T, _HD), lambda h, i: (h // rep, 0, 0)),
        ],
        out_specs=pl.BlockSpec((1, _BQ, _HD), lambda h, i: (h, i, 0)),
        out_shape=jax.ShapeDtypeStruct((_NH, T, _HD), jnp.float32),
    )(cos_t, sin_t, cos_t, sin_t, q, k, v)

    ctx2 = ctx.transpose(1, 0, 2).reshape(T, nq)

    res2, h2, we = pl.pallas_call(
        _post_kernel,
        grid=(nR,),
        in_specs=[
            pl.BlockSpec((_BS, nq), lambda r: (r, 0)),
            pl.BlockSpec((_BS, _H), lambda r: (r, 0)),
            pl.BlockSpec((1, _H), lambda r: (0, 0)),
            pl.BlockSpec((nq, _H), lambda r: (0, 0)),
            pl.BlockSpec((_H, _E), lambda r: (0, 0)),
            pl.BlockSpec((1, _E), lambda r: (0, 0)),
        ],
        out_specs=[
            pl.BlockSpec((_BS, _H), lambda r: (r, 0)),
            pl.BlockSpec((_BS, _H), lambda r: (r, 0)),
            pl.BlockSpec((_BS, _E), lambda r: (r, 0)),
        ],
        out_shape=[
            jax.ShapeDtypeStruct((T, _H), jnp.float32),
            jax.ShapeDtypeStruct((T, _H), jnp.bfloat16),
            jax.ShapeDtypeStruct((T, _E), jnp.float32),
        ],
    )(ctx2, res1, w_post_ln.reshape(1, _H), W_o, gate_w.T, gate_bias)

    shared = pl.pallas_call(
        _shared_kernel,
        grid=(nR,),
        in_specs=[
            pl.BlockSpec((_BS, _H), lambda r: (r, 0)),
            pl.BlockSpec((_H, 2 * _ISH), lambda r: (0, 0)),
            pl.BlockSpec((_ISH, _H), lambda r: (0, 0)),
        ],
        out_specs=pl.BlockSpec((_BS, _H), lambda r: (r, 0)),
        out_shape=jax.ShapeDtypeStruct((T, _H), jnp.float32),
    )(h2, sh_gu.astype(jnp.bfloat16), sh_dn.astype(jnp.bfloat16))

    h_out = pl.pallas_call(
        _moe_kernel,
        grid=(_E, nR),
        in_specs=[
            pl.BlockSpec((_BS, _H), lambda e, r: (r, 0)),
            pl.BlockSpec((_BS, _E), lambda e, r: (r, 0)),
            pl.BlockSpec((_BS, _H), lambda e, r: (r, 0)),
            pl.BlockSpec((1, _H, 2 * _IM), lambda e, r: (e, 0, 0)),
            pl.BlockSpec((1, _IM, _H), lambda e, r: (e, 0, 0)),
        ],
        out_specs=pl.BlockSpec((T, _H), lambda e, r: (0, 0)),
        out_shape=jax.ShapeDtypeStruct((T, _H), jnp.float32),
    )(h2, we, shared, exp_gu.astype(jnp.bfloat16), exp_dn.astype(jnp.bfloat16))

    return h_out, res2


# speedup vs baseline: 1.5165x; 1.5165x over previous
import functools

import numpy as np
import jax
import jax.numpy as jnp
from jax.experimental import pallas as pl
from jax.experimental.pallas import tpu as pltpu

_H = 1024
_NH = 16
_NKV = 4
_HD = 64
_E = 8
_TOPK = 2
_IM = 512
_ISH = 1024
_THETA = 500000.0
_EPS = 1e-06

_BS = 256
_BQ = 256
_BK = 256

_NQ = _NH * _HD
_NKVD = _NKV * _HD
_QH = _NQ // 2
_KH = _NKVD // 2


def _rot(x, cos, sin):
    n = cos.shape[1]
    x1 = x[:, :n]
    x2 = x[:, n:]
    return jnp.concatenate([x1 * cos - x2 * sin, x1 * sin + x2 * cos], axis=1)


def _pre_kernel(hid_ref, res_ref, wln_ref, wqkv_ref, cq_ref, sq_ref, ck_ref,
                sk_ref, res1_ref, qkv_ref):
    h = hid_ref[...] + res_ref[...]
    res1_ref[...] = h
    v = jnp.mean(h * h, axis=1, keepdims=True)
    ln = h * jax.lax.rsqrt(v + _EPS) * wln_ref[...]
    qkv = jnp.dot(ln, wqkv_ref[...], preferred_element_type=jnp.float32)
    qr = _rot(qkv[:, :_NQ], cq_ref[...], sq_ref[...])
    kr = _rot(qkv[:, _NQ:_NQ + _NKVD], ck_ref[...], sk_ref[...])
    qkv_ref[...] = jnp.concatenate([qr, kr, qkv[:, _NQ + _NKVD:]], axis=1)


def _attn_kernel(q_ref, k_ref, v_ref, o_ref, s_ref):
    i = pl.program_id(1)
    q = q_ref[0].reshape(4 * _BQ, _HD) * (_HD ** -0.5)

    def body1(j, m):
        kj = k_ref[0, pl.ds(j * _BK, _BK), :]
        s = jax.lax.dot_general(q, kj, (((1,), (1,)), ((), ())),
                                preferred_element_type=jnp.float32)
        row = jax.lax.broadcasted_iota(jnp.int32, (4 * _BQ, _BK), 0)
        qpos = i * _BQ + (row & (_BQ - 1))
        kpos = j * _BK + jax.lax.broadcasted_iota(jnp.int32, (4 * _BQ, _BK), 1)
        s = jnp.where(qpos >= kpos, s, -1e30)
        s_ref[:, pl.ds(j * _BK, _BK)] = s
        return jnp.maximum(m, jnp.max(s, axis=1, keepdims=True))

    m = jax.lax.fori_loop(0, i + 1, body1,
                          jnp.full((4 * _BQ, 1), -1e30, jnp.float32))

    def body2(j, l):
        cols = pl.ds(j * _BK, _BK)
        p = jnp.exp(s_ref[:, cols] - m)
        s_ref[:, cols] = p
        return l + jnp.sum(p, axis=1, keepdims=True)

    l = jax.lax.fori_loop(0, i + 1, body2,
                          jnp.zeros((4 * _BQ, 1), jnp.float32))
    rinv = 1.0 / l

    def body3(j, acc):
        cols = pl.ds(j * _BK, _BK)
        pn = s_ref[:, cols] * rinv
        vj = v_ref[0, cols, :]
        return acc + jnp.dot(pn, vj, preferred_element_type=jnp.float32)

    acc = jax.lax.fori_loop(0, i + 1, body3,
                            jnp.zeros((4 * _BQ, _HD), jnp.float32))
    o_ref[0] = acc.reshape(4, _BQ, _HD)


def _post_kernel(ctx_ref, res1_ref, wln_ref, wo_ref, gwt_ref, gb_ref,
                 res2_ref, h2_ref, we_ref):
    att = jnp.dot(ctx_ref[...], wo_ref[...], preferred_element_type=jnp.float32)
    r2 = att + res1_ref[...]
    res2_ref[...] = r2
    v = jnp.mean(r2 * r2, axis=1, keepdims=True)
    h2 = r2 * jax.lax.rsqrt(v + _EPS) * wln_ref[...]
    h2_ref[...] = h2.astype(jnp.bfloat16)
    logits = jnp.dot(h2, gwt_ref[...], preferred_element_type=jnp.float32)
    mx = jnp.max(logits, axis=1, keepdims=True)
    ex = jnp.exp(logits - mx)
    probs = ex / jnp.sum(ex, axis=1, keepdims=True)
    b = probs + gb_ref[...]
    idx = jax.lax.broadcasted_iota(jnp.int32, (_BS, _E), 1)
    m1 = jnp.max(b, axis=1, keepdims=True)
    a1 = jnp.min(jnp.where(b == m1, idx, _E), axis=1, keepdims=True)
    oh1 = idx == a1
    b2 = jnp.where(oh1, -1e30, b)
    m2 = jnp.max(b2, axis=1, keepdims=True)
    a2 = jnp.min(jnp.where(b2 == m2, idx, _E), axis=1, keepdims=True)
    sel = oh1 | (idx == a2)
    w = jnp.where(sel, probs, 0.0)
    we_ref[...] = w / jnp.sum(w, axis=1, keepdims=True)


def _shared_kernel(h2_ref, gu_ref, dn_ref, out_ref):
    g = jnp.dot(h2_ref[...], gu_ref[...], preferred_element_type=jnp.float32)
    g1 = g[:, :_ISH]
    g2 = g[:, _ISH:]
    a = (g1 * jax.nn.sigmoid(g1) * g2).astype(jnp.bfloat16)
    out_ref[...] = jnp.dot(a, dn_ref[...], preferred_element_type=jnp.float32)


def _moe_kernel(h2_ref, we_ref, shared_ref, gu_ref, dn_ref, out_ref):
    e = pl.program_id(0)
    r = pl.program_id(1)
    g = jnp.dot(h2_ref[...], gu_ref[0], preferred_element_type=jnp.float32)
    g1 = g[:, :_IM]
    g2 = g[:, _IM:]
    a = (g1 * jax.nn.sigmoid(g1) * g2).astype(jnp.bfloat16)
    xe = jnp.dot(a, dn_ref[0], preferred_element_type=jnp.float32)
    idx = jax.lax.broadcasted_iota(jnp.int32, (_BS, _E), 1)
    w = jnp.sum(we_ref[...] * (idx == e).astype(jnp.float32), axis=1,
                keepdims=True)
    contrib = w * xe
    rows = pl.ds(r * _BS, _BS)

    @pl.when(e == 0)
    def _():
        out_ref[rows, :] = shared_ref[...] + contrib

    @pl.when(e != 0)
    def _():
        out_ref[rows, :] = out_ref[rows, :] + contrib


def kernel(hidden_states, residual, w_in_ln, W_qkv, W_o, w_post_ln, gate_w,
           gate_bias, exp_gu, exp_dn, sh_gu, sh_dn, positions):
    T = hidden_states.shape[0]

    def half_perm(nheads, base):
        even = (np.arange(nheads)[:, None] * _HD
                + np.arange(0, _HD, 2)[None, :]).reshape(-1)
        return np.concatenate([base + even, base + even + 1])

    perm = np.concatenate([half_perm(_NH, 0), half_perm(_NKV, _NQ),
                           np.arange(_NQ + _NKVD, _NQ + 2 * _NKVD)])
    wqkv_p = W_qkv[:, perm]

    inv_freq = 1.0 / (_THETA ** (jnp.arange(0, _HD, 2, dtype=jnp.float32) / _HD))
    freqs = positions.astype(jnp.float32)[:, None] * inv_freq[None, :]
    cos_t = jnp.cos(freqs)
    sin_t = jnp.sin(freqs)
    cq = jnp.tile(cos_t, (1, _NH))
    sq = jnp.tile(sin_t, (1, _NH))
    ck = jnp.tile(cos_t, (1, _NKV))
    sk = jnp.tile(sin_t, (1, _NKV))

    nR = T // _BS

    res1, qkv = pl.pallas_call(
        _pre_kernel,
        grid=(nR,),
        in_specs=[
            pl.BlockSpec((_BS, _H), lambda r: (r, 0)),
            pl.BlockSpec((_BS, _H), lambda r: (r, 0)),
            pl.BlockSpec((1, _H), lambda r: (0, 0)),
            pl.BlockSpec((_H, _NQ + 2 * _NKVD), lambda r: (0, 0)),
            pl.BlockSpec((_BS, _QH), lambda r: (r, 0)),
            pl.BlockSpec((_BS, _QH), lambda r: (r, 0)),
            pl.BlockSpec((_BS, _KH), lambda r: (r, 0)),
            pl.BlockSpec((_BS, _KH), lambda r: (r, 0)),
        ],
        out_specs=[
            pl.BlockSpec((_BS, _H), lambda r: (r, 0)),
            pl.BlockSpec((_BS, _NQ + 2 * _NKVD), lambda r: (r, 0)),
        ],
        out_shape=[
            jax.ShapeDtypeStruct((T, _H), jnp.float32),
            jax.ShapeDtypeStruct((T, _NQ + 2 * _NKVD), jnp.float32),
        ],
    )(hidden_states, residual, w_in_ln.reshape(1, _H), wqkv_p, cq, sq, ck, sk)

    q = (qkv[:, :_NQ].reshape(T, 2, _NH, _HD // 2)
         .transpose(2, 1, 0, 3).reshape(_NKV, 4, 2, T, _HD // 2)
         .transpose(0, 1, 3, 2, 4).reshape(_NKV, 4, T, _HD))
    k = (qkv[:, _NQ:_NQ + _NKVD].reshape(T, 2, _NKV, _HD // 2)
         .transpose(2, 0, 1, 3).reshape(_NKV, T, _HD))
    v = (qkv[:, _NQ + _NKVD:].reshape(T, _NKV, _HD)
         .transpose(1, 0, 2))

    ctx4 = pl.pallas_call(
        _attn_kernel,
        grid=(_NKV, T // _BQ),
        in_specs=[
            pl.BlockSpec((1, 4, _BQ, _HD), lambda g, i: (g, 0, i, 0)),
            pl.BlockSpec((1, T, _HD), lambda g, i: (g, 0, 0)),
            pl.BlockSpec((1, T, _HD), lambda g, i: (g, 0, 0)),
        ],
        out_specs=pl.BlockSpec((1, 4, _BQ, _HD), lambda g, i: (g, 0, i, 0)),
        out_shape=jax.ShapeDtypeStruct((_NKV, 4, T, _HD), jnp.float32),
        scratch_shapes=[pltpu.VMEM((4 * _BQ, T), jnp.float32)],
    )(q, k, v)

    ctx2 = ctx4.reshape(_NH, T, _HD).transpose(1, 0, 2).reshape(T, _NQ)

    res2, h2, we = pl.pallas_call(
        _post_kernel,
        grid=(nR,),
        in_specs=[
            pl.BlockSpec((_BS, _NQ), lambda r: (r, 0)),
            pl.BlockSpec((_BS, _H), lambda r: (r, 0)),
            pl.BlockSpec((1, _H), lambda r: (0, 0)),
            pl.BlockSpec((_NQ, _H), lambda r: (0, 0)),
            pl.BlockSpec((_H, _E), lambda r: (0, 0)),
            pl.BlockSpec((1, _E), lambda r: (0, 0)),
        ],
        out_specs=[
            pl.BlockSpec((_BS, _H), lambda r: (r, 0)),
            pl.BlockSpec((_BS, _H), lambda r: (r, 0)),
            pl.BlockSpec((_BS, _E), lambda r: (r, 0)),
        ],
        out_shape=[
            jax.ShapeDtypeStruct((T, _H), jnp.float32),
            jax.ShapeDtypeStruct((T, _H), jnp.bfloat16),
            jax.ShapeDtypeStruct((T, _E), jnp.float32),
        ],
    )(ctx2, res1, w_post_ln.reshape(1, _H), W_o, gate_w.T, gate_bias)

    shared = pl.pallas_call(
        _shared_kernel,
        grid=(nR,),
        in_specs=[
            pl.BlockSpec((_BS, _H), lambda r: (r, 0)),
            pl.BlockSpec((_H, 2 * _ISH), lambda r: (0, 0)),
            pl.BlockSpec((_ISH, _H), lambda r: (0, 0)),
        ],
        out_specs=pl.BlockSpec((_BS, _H), lambda r: (r, 0)),
        out_shape=jax.ShapeDtypeStruct((T, _H), jnp.float32),
    )(h2, sh_gu.astype(jnp.bfloat16), sh_dn.astype(jnp.bfloat16))

    h_out = pl.pallas_call(
        _moe_kernel,
        grid=(_E, nR),
        in_specs=[
            pl.BlockSpec((_BS, _H), lambda e, r: (r, 0)),
            pl.BlockSpec((_BS, _E), lambda e, r: (r, 0)),
            pl.BlockSpec((_BS, _H), lambda e, r: (r, 0)),
            pl.BlockSpec((1, _H, 2 * _IM), lambda e, r: (e, 0, 0)),
            pl.BlockSpec((1, _IM, _H), lambda e, r: (e, 0, 0)),
        ],
        out_specs=pl.BlockSpec((T, _H), lambda e, r: (0, 0)),
        out_shape=jax.ShapeDtypeStruct((T, _H), jnp.float32),
    )(h2, we, shared, exp_gu.astype(jnp.bfloat16), exp_dn.astype(jnp.bfloat16))

    return h_out, res2


# speedup vs baseline: 1.8037x; 1.1894x over previous
import numpy as np
import jax
import jax.numpy as jnp
from jax.experimental import pallas as pl
from jax.experimental.pallas import tpu as pltpu

_H = 1024
_NH = 16
_NKV = 4
_HD = 64
_E = 8
_IM = 512
_ISH = 1024
_THETA = 500000.0
_EPS = 1e-06

_BS = 256
_BQ = 256
_BK = 256

_NQ = _NH * _HD
_NKVD = _NKV * _HD
_QKVW = _NQ + 2 * _NKVD


def _rope_head(x, cos, sin):
    x1 = x[:, :_HD // 2]
    x2 = x[:, _HD // 2:]
    return jnp.concatenate([x1 * cos - x2 * sin, x1 * sin + x2 * cos], axis=1)


def _pre_kernel(hid_ref, res_ref, wln_ref, wqkv_ref, c_ref, s_ref,
                res1_ref, q_ref, k_ref, v_ref):
    h = hid_ref[...] + res_ref[...]
    res1_ref[...] = h
    var = jnp.mean(h * h, axis=1, keepdims=True)
    ln = h * jax.lax.rsqrt(var + _EPS) * wln_ref[...]
    qkv = jnp.dot(ln, wqkv_ref[...], preferred_element_type=jnp.float32)
    c = c_ref[...]
    s = s_ref[...]
    q_ref[...] = jnp.concatenate(
        [_rope_head(qkv[:, h0:h0 + _HD], c, s)
         for h0 in range(0, _NQ, _HD)], axis=1)
    k_ref[...] = jnp.stack(
        [_rope_head(qkv[:, h0:h0 + _HD], c, s)
         for h0 in range(_NQ, _NQ + _NKVD, _HD)], axis=0)
    v_ref[...] = jnp.stack(
        [qkv[:, h0:h0 + _HD]
         for h0 in range(_NQ + _NKVD, _QKVW, _HD)], axis=0)


def _attn_kernel(q_ref, k_ref, v_ref, o_ref, s_ref):
    i = pl.program_id(1)
    qb = q_ref[...]
    q = jnp.concatenate([qb[:, h0:h0 + _HD]
                         for h0 in range(0, 4 * _HD, _HD)], axis=0) * 0.125

    def body1(j, m):
        kj = k_ref[0, pl.ds(j * _BK, _BK), :]
        s = jax.lax.dot_general(q, kj, (((1,), (1,)), ((), ())),
                                preferred_element_type=jnp.float32)
        row = jax.lax.broadcasted_iota(jnp.int32, (4 * _BQ, _BK), 0)
        qpos = i * _BQ + (row & (_BQ - 1))
        kpos = j * _BK + jax.lax.broadcasted_iota(jnp.int32, (4 * _BQ, _BK), 1)
        s = jnp.where(qpos >= kpos, s, -1e30)
        s_ref[:, pl.ds(j * _BK, _BK)] = s
        return jnp.maximum(m, jnp.max(s, axis=1, keepdims=True))

    m = jax.lax.fori_loop(0, i + 1, body1,
                          jnp.full((4 * _BQ, 1), -1e30, jnp.float32))

    def body2(j, l):
        cols = pl.ds(j * _BK, _BK)
        p = jnp.exp(s_ref[:, cols] - m)
        s_ref[:, cols] = p
        return l + jnp.sum(p, axis=1, keepdims=True)

    l = jax.lax.fori_loop(0, i + 1, body2,
                          jnp.zeros((4 * _BQ, 1), jnp.float32))
    rinv = 1.0 / l

    def body3(j, acc):
        cols = pl.ds(j * _BK, _BK)
        pn = s_ref[:, cols] * rinv
        vj = v_ref[0, pl.ds(j * _BK, _BK), :]
        return acc + jnp.dot(pn, vj, preferred_element_type=jnp.float32)

    acc = jax.lax.fori_loop(0, i + 1, body3,
                            jnp.zeros((4 * _BQ, _HD), jnp.float32))
    o_ref[...] = jnp.concatenate([acc[h0 * _BQ:(h0 + 1) * _BQ, :]
                                  for h0 in range(4)], axis=1)


def _post_kernel(ctx_ref, res1_ref, wln_ref, wo_ref, gwt_ref, gb_ref,
                 sgu_ref, sdn_ref, res2_ref, h2_ref, we_ref, shared_ref):
    att = jnp.dot(ctx_ref[...], wo_ref[...], preferred_element_type=jnp.float32)
    r2 = att + res1_ref[...]
    res2_ref[...] = r2
    var = jnp.mean(r2 * r2, axis=1, keepdims=True)
    h2 = r2 * jax.lax.rsqrt(var + _EPS) * wln_ref[...]
    h2b = h2.astype(jnp.bfloat16)
    h2_ref[...] = h2b
    logits = jnp.dot(h2, gwt_ref[...], preferred_element_type=jnp.float32)
    mx = jnp.max(logits, axis=1, keepdims=True)
    ex = jnp.exp(logits - mx)
    probs = ex / jnp.sum(ex, axis=1, keepdims=True)
    b = probs + gb_ref[...]
    idx = jax.lax.broadcasted_iota(jnp.int32, (_BS, _E), 1)
    m1 = jnp.max(b, axis=1, keepdims=True)
    a1 = jnp.min(jnp.where(b == m1, idx, _E), axis=1, keepdims=True)
    oh1 = idx == a1
    b2 = jnp.where(oh1, -1e30, b)
    m2 = jnp.max(b2, axis=1, keepdims=True)
    a2 = jnp.min(jnp.where(b2 == m2, idx, _E), axis=1, keepdims=True)
    sel = oh1 | (idx == a2)
    w = jnp.where(sel, probs, 0.0)
    we_ref[...] = w / jnp.sum(w, axis=1, keepdims=True)
    g = jnp.dot(h2b, sgu_ref[...].astype(jnp.bfloat16),
                preferred_element_type=jnp.float32)
    g1 = g[:, :_ISH]
    g2 = g[:, _ISH:]
    a = (g1 * jax.nn.sigmoid(g1) * g2).astype(jnp.bfloat16)
    shared_ref[...] = jnp.dot(a, sdn_ref[...].astype(jnp.bfloat16),
                              preferred_element_type=jnp.float32)


def _moe_kernel(h2_ref, we_ref, shared_ref, gu_ref, dn_ref, out_ref):
    e = pl.program_id(0)
    r = pl.program_id(1)
    g = jnp.dot(h2_ref[...], gu_ref[0].astype(jnp.bfloat16),
                preferred_element_type=jnp.float32)
    g1 = g[:, :_IM]
    g2 = g[:, _IM:]
    a = (g1 * jax.nn.sigmoid(g1) * g2).astype(jnp.bfloat16)
    xe = jnp.dot(a, dn_ref[0].astype(jnp.bfloat16),
                 preferred_element_type=jnp.float32)
    idx = jax.lax.broadcasted_iota(jnp.int32, (_BS, _E), 1)
    w = jnp.sum(we_ref[...] * (idx == e).astype(jnp.float32), axis=1,
                keepdims=True)
    contrib = w * xe
    rows = pl.ds(r * _BS, _BS)

    @pl.when(e == 0)
    def _():
        out_ref[rows, :] = shared_ref[...] + contrib

    @pl.when(e != 0)
    def _():
        out_ref[rows, :] = out_ref[rows, :] + contrib


def kernel(hidden_states, residual, w_in_ln, W_qkv, W_o, w_post_ln, gate_w,
           gate_bias, exp_gu, exp_dn, sh_gu, sh_dn, positions):
    T = hidden_states.shape[0]

    head = np.concatenate([np.arange(0, _HD, 2), np.arange(1, _HD, 2)])
    perm = np.concatenate(
        [h0 + head for h0 in range(0, _NQ + _NKVD, _HD)]
        + [np.arange(_NQ + _NKVD, _QKVW)])
    wqkv_p = W_qkv[:, perm]

    inv_freq = 1.0 / (_THETA ** (jnp.arange(0, _HD, 2, dtype=jnp.float32) / _HD))
    freqs = positions.astype(jnp.float32)[:, None] * inv_freq[None, :]
    cos_t = jnp.cos(freqs)
    sin_t = jnp.sin(freqs)

    nR = T // _BS

    res1, q, k, v = pl.pallas_call(
        _pre_kernel,
        grid=(nR,),
        in_specs=[
            pl.BlockSpec((_BS, _H), lambda r: (r, 0)),
            pl.BlockSpec((_BS, _H), lambda r: (r, 0)),
            pl.BlockSpec((1, _H), lambda r: (0, 0)),
            pl.BlockSpec((_H, _QKVW), lambda r: (0, 0)),
            pl.BlockSpec((_BS, _HD // 2), lambda r: (r, 0)),
            pl.BlockSpec((_BS, _HD // 2), lambda r: (r, 0)),
        ],
        out_specs=[
            pl.BlockSpec((_BS, _H), lambda r: (r, 0)),
            pl.BlockSpec((_BS, _NQ), lambda r: (r, 0)),
            pl.BlockSpec((_NKV, _BS, _HD), lambda r: (0, r, 0)),
            pl.BlockSpec((_NKV, _BS, _HD), lambda r: (0, r, 0)),
        ],
        out_shape=[
            jax.ShapeDtypeStruct((T, _H), jnp.float32),
            jax.ShapeDtypeStruct((T, _NQ), jnp.float32),
            jax.ShapeDtypeStruct((_NKV, T, _HD), jnp.float32),
            jax.ShapeDtypeStruct((_NKV, T, _HD), jnp.float32),
        ],
    )(hidden_states, residual, w_in_ln.reshape(1, _H), wqkv_p, cos_t, sin_t)

    ctx = pl.pallas_call(
        _attn_kernel,
        grid=(_NKV, T // _BQ),
        in_specs=[
            pl.BlockSpec((_BQ, 4 * _HD), lambda g, i: (i, g)),
            pl.BlockSpec((1, T, _HD), lambda g, i: (g, 0, 0)),
            pl.BlockSpec((1, T, _HD), lambda g, i: (g, 0, 0)),
        ],
        out_specs=pl.BlockSpec((_BQ, 4 * _HD), lambda g, i: (i, g)),
        out_shape=jax.ShapeDtypeStruct((T, _NQ), jnp.float32),
        scratch_shapes=[pltpu.VMEM((4 * _BQ, T), jnp.float32)],
    )(q, k, v)

    res2, h2, we, shared = pl.pallas_call(
        _post_kernel,
        grid=(nR,),
        in_specs=[
            pl.BlockSpec((_BS, _NQ), lambda r: (r, 0)),
            pl.BlockSpec((_BS, _H), lambda r: (r, 0)),
            pl.BlockSpec((1, _H), lambda r: (0, 0)),
            pl.BlockSpec((_NQ, _H), lambda r: (0, 0)),
            pl.BlockSpec((_H, _E), lambda r: (0, 0)),
            pl.BlockSpec((1, _E), lambda r: (0, 0)),
            pl.BlockSpec((_H, 2 * _ISH), lambda r: (0, 0)),
            pl.BlockSpec((_ISH, _H), lambda r: (0, 0)),
        ],
        out_specs=[
            pl.BlockSpec((_BS, _H), lambda r: (r, 0)),
            pl.BlockSpec((_BS, _H), lambda r: (r, 0)),
            pl.BlockSpec((_BS, _E), lambda r: (r, 0)),
            pl.BlockSpec((_BS, _H), lambda r: (r, 0)),
        ],
        out_shape=[
            jax.ShapeDtypeStruct((T, _H), jnp.float32),
            jax.ShapeDtypeStruct((T, _H), jnp.bfloat16),
            jax.ShapeDtypeStruct((T, _E), jnp.float32),
            jax.ShapeDtypeStruct((T, _H), jnp.float32),
        ],
    )(ctx, res1, w_post_ln.reshape(1, _H), W_o, gate_w.T, gate_bias,
      sh_gu, sh_dn)

    h_out = pl.pallas_call(
        _moe_kernel,
        grid=(_E, nR),
        in_specs=[
            pl.BlockSpec((_BS, _H), lambda e, r: (r, 0)),
            pl.BlockSpec((_BS, _E), lambda e, r: (r, 0)),
            pl.BlockSpec((_BS, _H), lambda e, r: (r, 0)),
            pl.BlockSpec((1, _H, 2 * _IM), lambda e, r: (e, 0, 0)),
            pl.BlockSpec((1, _IM, _H), lambda e, r: (e, 0, 0)),
        ],
        out_specs=pl.BlockSpec((T, _H), lambda e, r: (0, 0)),
        out_shape=jax.ShapeDtypeStruct((T, _H), jnp.float32),
    )(h2, we, shared, exp_gu, exp_dn)

    return h_out, res2


# speedup vs baseline: 1.9163x; 1.0624x over previous
import numpy as np
import jax
import jax.numpy as jnp
from jax.experimental import pallas as pl
from jax.experimental.pallas import tpu as pltpu

_H = 1024
_NH = 16
_NKV = 4
_HD = 64
_E = 8
_IM = 512
_ISH = 1024
_THETA = 500000.0
_EPS = 1e-06

_BS = 256
_BQ = 256
_BK = 256

_NQ = _NH * _HD
_NKVD = _NKV * _HD
_QKVW = _NQ + 2 * _NKVD


def _rope_head(x, cos, sin):
    x1 = x[:, :_HD // 2]
    x2 = x[:, _HD // 2:]
    return jnp.concatenate([x1 * cos - x2 * sin, x1 * sin + x2 * cos], axis=1)


def _pre_kernel(hid_ref, res_ref, wln_ref, wqkv_ref, c_ref, s_ref,
                res1_ref, q_ref, k_ref, v_ref):
    h = hid_ref[...] + res_ref[...]
    res1_ref[...] = h
    var = jnp.mean(h * h, axis=1, keepdims=True)
    ln = h * jax.lax.rsqrt(var + _EPS) * wln_ref[...]
    qkv = jnp.dot(ln, wqkv_ref[...], preferred_element_type=jnp.float32)
    c = c_ref[...]
    s = s_ref[...]
    q_ref[...] = jnp.concatenate(
        [_rope_head(qkv[:, h0:h0 + _HD], c, s)
         for h0 in range(0, _NQ, _HD)], axis=1)
    k_ref[...] = jnp.stack(
        [_rope_head(qkv[:, h0:h0 + _HD], c, s)
         for h0 in range(_NQ, _NQ + _NKVD, _HD)], axis=0)
    v_ref[...] = jnp.stack(
        [qkv[:, h0:h0 + _HD]
         for h0 in range(_NQ + _NKVD, _QKVW, _HD)], axis=0)


def _attn_kernel(q_ref, k_ref, v_ref, o_ref, s_ref):
    i = pl.program_id(1)
    qb = q_ref[...]
    q = jnp.concatenate([qb[:, h0:h0 + _HD]
                         for h0 in range(0, 4 * _HD, _HD)], axis=0) * 0.125
    row = jax.lax.broadcasted_iota(jnp.int32, (4 * _BQ, _BK), 0)
    qpos = i * _BQ + (row & (_BQ - 1))
    col = jax.lax.broadcasted_iota(jnp.int32, (4 * _BQ, _BK), 1)

    def body1(j, m):
        kj = k_ref[0, pl.ds(j * _BK, _BK), :]
        s = jax.lax.dot_general(q, kj, (((1,), (1,)), ((), ())),
                                preferred_element_type=jnp.float32)
        s = jnp.where(qpos >= col + j * _BK, s, -1e30)
        s_ref[:, pl.ds(j * _BK, _BK)] = s
        return jnp.maximum(m, jnp.max(s, axis=1, keepdims=True))

    m = jax.lax.fori_loop(0, i + 1, body1,
                          jnp.full((4 * _BQ, 1), -1e30, jnp.float32))

    def body2(j, carry):
        acc, l = carry
        cols = pl.ds(j * _BK, _BK)
        p = jnp.exp(s_ref[:, cols] - m)
        vj = v_ref[0, cols, :]
        l = l + jnp.sum(p, axis=1, keepdims=True)
        acc = acc + jnp.dot(p, vj, preferred_element_type=jnp.float32)
        return acc, l

    acc, l = jax.lax.fori_loop(
        0, i + 1, body2,
        (jnp.zeros((4 * _BQ, _HD), jnp.float32),
         jnp.zeros((4 * _BQ, 1), jnp.float32)))
    acc = acc * (1.0 / l)
    o_ref[...] = jnp.concatenate([acc[h0 * _BQ:(h0 + 1) * _BQ, :]
                                  for h0 in range(4)], axis=1)


def _post_kernel(ctx_ref, res1_ref, wln_ref, wo_ref, gwt_ref, gb_ref,
                 sgu_ref, sdn_ref, res2_ref, h2_ref, we_ref, shared_ref):
    att = jnp.dot(ctx_ref[...], wo_ref[...], preferred_element_type=jnp.float32)
    r2 = att + res1_ref[...]
    res2_ref[...] = r2
    var = jnp.mean(r2 * r2, axis=1, keepdims=True)
    h2 = r2 * jax.lax.rsqrt(var + _EPS) * wln_ref[...]
    h2b = h2.astype(jnp.bfloat16)
    h2_ref[...] = h2b
    logits = jnp.dot(h2, gwt_ref[...], preferred_element_type=jnp.float32)
    mx = jnp.max(logits, axis=1, keepdims=True)
    ex = jnp.exp(logits - mx)
    probs = ex / jnp.sum(ex, axis=1, keepdims=True)
    b = probs + gb_ref[...]
    idx = jax.lax.broadcasted_iota(jnp.int32, (_BS, _E), 1)
    m1 = jnp.max(b, axis=1, keepdims=True)
    a1 = jnp.min(jnp.where(b == m1, idx, _E), axis=1, keepdims=True)
    oh1 = idx == a1
    b2 = jnp.where(oh1, -1e30, b)
    m2 = jnp.max(b2, axis=1, keepdims=True)
    a2 = jnp.min(jnp.where(b2 == m2, idx, _E), axis=1, keepdims=True)
    sel = oh1 | (idx == a2)
    w = jnp.where(sel, probs, 0.0)
    we_ref[...] = w / jnp.sum(w, axis=1, keepdims=True)
    g = jnp.dot(h2b, sgu_ref[...].astype(jnp.bfloat16),
                preferred_element_type=jnp.float32)
    g1 = g[:, :_ISH]
    g2 = g[:, _ISH:]
    a = (g1 * jax.nn.sigmoid(g1) * g2).astype(jnp.bfloat16)
    shared_ref[...] = jnp.dot(a, sdn_ref[...].astype(jnp.bfloat16),
                              preferred_element_type=jnp.float32)


def _moe_kernel(h2_ref, we_ref, shared_ref, gu_ref, dn_ref, out_ref):
    e = pl.program_id(0)
    r = pl.program_id(1)
    g = jnp.dot(h2_ref[...], gu_ref[0].astype(jnp.bfloat16),
                preferred_element_type=jnp.float32)
    g1 = g[:, :_IM]
    g2 = g[:, _IM:]
    a = (g1 * jax.nn.sigmoid(g1) * g2).astype(jnp.bfloat16)
    xe = jnp.dot(a, dn_ref[0].astype(jnp.bfloat16),
                 preferred_element_type=jnp.float32)
    idx = jax.lax.broadcasted_iota(jnp.int32, (_BS, _E), 1)
    w = jnp.sum(we_ref[...] * (idx == e).astype(jnp.float32), axis=1,
                keepdims=True)
    contrib = w * xe
    rows = pl.ds(r * _BS, _BS)

    @pl.when(e == 0)
    def _():
        out_ref[rows, :] = shared_ref[...] + contrib

    @pl.when(e != 0)
    def _():
        out_ref[rows, :] = out_ref[rows, :] + contrib


def kernel(hidden_states, residual, w_in_ln, W_qkv, W_o, w_post_ln, gate_w,
           gate_bias, exp_gu, exp_dn, sh_gu, sh_dn, positions):
    T = hidden_states.shape[0]

    head = np.concatenate([np.arange(0, _HD, 2), np.arange(1, _HD, 2)])
    perm = np.concatenate(
        [h0 + head for h0 in range(0, _NQ + _NKVD, _HD)]
        + [np.arange(_NQ + _NKVD, _QKVW)])
    wqkv_p = W_qkv[:, perm]

    inv_freq = 1.0 / (_THETA ** (jnp.arange(0, _HD, 2, dtype=jnp.float32) / _HD))
    freqs = positions.astype(jnp.float32)[:, None] * inv_freq[None, :]
    cos_t = jnp.cos(freqs)
    sin_t = jnp.sin(freqs)

    nR = T // _BS

    res1, q, k, v = pl.pallas_call(
        _pre_kernel,
        grid=(nR,),
        in_specs=[
            pl.BlockSpec((_BS, _H), lambda r: (r, 0)),
            pl.BlockSpec((_BS, _H), lambda r: (r, 0)),
            pl.BlockSpec((1, _H), lambda r: (0, 0)),
            pl.BlockSpec((_H, _QKVW), lambda r: (0, 0)),
            pl.BlockSpec((_BS, _HD // 2), lambda r: (r, 0)),
            pl.BlockSpec((_BS, _HD // 2), lambda r: (r, 0)),
        ],
        out_specs=[
            pl.BlockSpec((_BS, _H), lambda r: (r, 0)),
            pl.BlockSpec((_BS, _NQ), lambda r: (r, 0)),
            pl.BlockSpec((_NKV, _BS, _HD), lambda r: (0, r, 0)),
            pl.BlockSpec((_NKV, _BS, _HD), lambda r: (0, r, 0)),
        ],
        out_shape=[
            jax.ShapeDtypeStruct((T, _H), jnp.float32),
            jax.ShapeDtypeStruct((T, _NQ), jnp.float32),
            jax.ShapeDtypeStruct((_NKV, T, _HD), jnp.float32),
            jax.ShapeDtypeStruct((_NKV, T, _HD), jnp.float32),
        ],
    )(hidden_states, residual, w_in_ln.reshape(1, _H), wqkv_p, cos_t, sin_t)

    ctx = pl.pallas_call(
        _attn_kernel,
        grid=(_NKV, T // _BQ),
        in_specs=[
            pl.BlockSpec((_BQ, 4 * _HD), lambda g, i: (i, g)),
            pl.BlockSpec((1, T, _HD), lambda g, i: (g, 0, 0)),
            pl.BlockSpec((1, T, _HD), lambda g, i: (g, 0, 0)),
        ],
        out_specs=pl.BlockSpec((_BQ, 4 * _HD), lambda g, i: (i, g)),
        out_shape=jax.ShapeDtypeStruct((T, _NQ), jnp.float32),
        scratch_shapes=[pltpu.VMEM((4 * _BQ, T), jnp.float32)],
    )(q, k, v)

    res2, h2, we, shared = pl.pallas_call(
        _post_kernel,
        grid=(nR,),
        in_specs=[
            pl.BlockSpec((_BS, _NQ), lambda r: (r, 0)),
            pl.BlockSpec((_BS, _H), lambda r: (r, 0)),
            pl.BlockSpec((1, _H), lambda r: (0, 0)),
            pl.BlockSpec((_NQ, _H), lambda r: (0, 0)),
            pl.BlockSpec((_H, _E), lambda r: (0, 0)),
            pl.BlockSpec((1, _E), lambda r: (0, 0)),
            pl.BlockSpec((_H, 2 * _ISH), lambda r: (0, 0)),
            pl.BlockSpec((_ISH, _H), lambda r: (0, 0)),
        ],
        out_specs=[
            pl.BlockSpec((_BS, _H), lambda r: (r, 0)),
            pl.BlockSpec((_BS, _H), lambda r: (r, 0)),
            pl.BlockSpec((_BS, _E), lambda r: (r, 0)),
            pl.BlockSpec((_BS, _H), lambda r: (r, 0)),
        ],
        out_shape=[
            jax.ShapeDtypeStruct((T, _H), jnp.float32),
            jax.ShapeDtypeStruct((T, _H), jnp.bfloat16),
            jax.ShapeDtypeStruct((T, _E), jnp.float32),
            jax.ShapeDtypeStruct((T, _H), jnp.float32),
        ],
    )(ctx, res1, w_post_ln.reshape(1, _H), W_o, gate_w.T, gate_bias,
      sh_gu, sh_dn)

    h_out = pl.pallas_call(
        _moe_kernel,
        grid=(_E, nR),
        in_specs=[
            pl.BlockSpec((_BS, _H), lambda e, r: (r, 0)),
            pl.BlockSpec((_BS, _E), lambda e, r: (r, 0)),
            pl.BlockSpec((_BS, _H), lambda e, r: (r, 0)),
            pl.BlockSpec((1, _H, 2 * _IM), lambda e, r: (e, 0, 0)),
            pl.BlockSpec((1, _IM, _H), lambda e, r: (e, 0, 0)),
        ],
        out_specs=pl.BlockSpec((T, _H), lambda e, r: (0, 0)),
        out_shape=jax.ShapeDtypeStruct((T, _H), jnp.float32),
    )(h2, we, shared, exp_gu, exp_dn)

    return h_out, res2


# speedup vs baseline: 2.2727x; 1.1860x over previous
import numpy as np
import jax
import jax.numpy as jnp
from jax.experimental import pallas as pl
from jax.experimental.pallas import tpu as pltpu

_H = 1024
_NH = 16
_NKV = 4
_HD = 64
_E = 8
_IM = 512
_ISH = 1024
_THETA = 500000.0
_EPS = 1e-06

_BS = 256
_BQ = 256
_BK = 512
_BSM = 512

_NQ = _NH * _HD
_NKVD = _NKV * _HD
_QKVW = _NQ + 2 * _NKVD


def _rope_head(x, cos, sin):
    x1 = x[:, :_HD // 2]
    x2 = x[:, _HD // 2:]
    return jnp.concatenate([x1 * cos - x2 * sin, x1 * sin + x2 * cos], axis=1)


def _pre_kernel(hid_ref, res_ref, wln_ref, wqkv_ref, c_ref, s_ref,
                res1_ref, q_ref, k_ref, v_ref):
    h = hid_ref[...] + res_ref[...]
    res1_ref[...] = h
    var = jnp.mean(h * h, axis=1, keepdims=True)
    ln = h * jax.lax.rsqrt(var + _EPS) * wln_ref[...]
    qkv = jnp.dot(ln, wqkv_ref[...], preferred_element_type=jnp.float32)
    c = c_ref[...]
    s = s_ref[...]
    q_ref[...] = jnp.concatenate(
        [_rope_head(qkv[:, h0:h0 + _HD], c, s)
         for h0 in range(0, _NQ, _HD)], axis=1)
    k_ref[...] = jnp.stack(
        [_rope_head(qkv[:, h0:h0 + _HD], c, s)
         for h0 in range(_NQ, _NQ + _NKVD, _HD)], axis=0)
    v_ref[...] = jnp.stack(
        [qkv[:, h0:h0 + _HD]
         for h0 in range(_NQ + _NKVD, _QKVW, _HD)], axis=0)


def _attn_kernel(q_ref, k_ref, v_ref, o_ref, s_ref):
    i = pl.program_id(1)
    qb = q_ref[...]
    q = jnp.concatenate([qb[:, h0:h0 + _HD]
                         for h0 in range(0, 4 * _HD, _HD)], axis=0) * 0.125
    row = jax.lax.broadcasted_iota(jnp.int32, (4 * _BQ, _BK), 0)
    qpos = i * _BQ + (row & (_BQ - 1))
    col = jax.lax.broadcasted_iota(jnp.int32, (4 * _BQ, _BK), 1)

    def body1(j, m):
        kj = k_ref[0, pl.ds(j * _BK, _BK), :]
        s = jax.lax.dot_general(q, kj, (((1,), (1,)), ((), ())),
                                preferred_element_type=jnp.float32)
        s = jnp.where(qpos >= col + j * _BK, s, -1e30)
        s_ref[:, pl.ds(j * _BK, _BK)] = s
        return jnp.maximum(m, jnp.max(s, axis=1, keepdims=True))

    nkb = ((i + 1) * _BQ + _BK - 1) // _BK
    m = jax.lax.fori_loop(0, nkb, body1,
                          jnp.full((4 * _BQ, 1), -1e30, jnp.float32))

    def body2(j, carry):
        acc, l = carry
        cols = pl.ds(j * _BK, _BK)
        p = jnp.exp(s_ref[:, cols] - m)
        vj = v_ref[0, cols, :]
        l = l + jnp.sum(p, axis=1, keepdims=True)
        acc = acc + jnp.dot(p, vj, preferred_element_type=jnp.float32)
        return acc, l

    acc, l = jax.lax.fori_loop(
        0, nkb, body2,
        (jnp.zeros((4 * _BQ, _HD), jnp.float32),
         jnp.zeros((4 * _BQ, 1), jnp.float32)))
    acc = acc * (1.0 / l)
    o_ref[...] = jnp.concatenate([acc[h0 * _BQ:(h0 + 1) * _BQ, :]
                                  for h0 in range(4)], axis=1)


def _post_kernel(ctx_ref, res1_ref, wln_ref, wo_ref, gwt_ref, gb_ref,
                 sgu_ref, sdn_ref, res2_ref, h2_ref, we_ref, shared_ref):
    att = jnp.dot(ctx_ref[...], wo_ref[...], preferred_element_type=jnp.float32)
    r2 = att + res1_ref[...]
    res2_ref[...] = r2
    var = jnp.mean(r2 * r2, axis=1, keepdims=True)
    h2 = r2 * jax.lax.rsqrt(var + _EPS) * wln_ref[...]
    h2b = h2.astype(jnp.bfloat16)
    h2_ref[...] = h2b
    logits = jnp.dot(h2, gwt_ref[...], preferred_element_type=jnp.float32)
    mx = jnp.max(logits, axis=1, keepdims=True)
    ex = jnp.exp(logits - mx)
    probs = ex / jnp.sum(ex, axis=1, keepdims=True)
    b = probs + gb_ref[...]
    idx = jax.lax.broadcasted_iota(jnp.int32, (_BS, _E), 1)
    m1 = jnp.max(b, axis=1, keepdims=True)
    a1 = jnp.min(jnp.where(b == m1, idx, _E), axis=1, keepdims=True)
    oh1 = idx == a1
    b2 = jnp.where(oh1, -1e30, b)
    m2 = jnp.max(b2, axis=1, keepdims=True)
    a2 = jnp.min(jnp.where(b2 == m2, idx, _E), axis=1, keepdims=True)
    sel = oh1 | (idx == a2)
    w = jnp.where(sel, probs, 0.0)
    we_ref[...] = w / jnp.sum(w, axis=1, keepdims=True)
    g = jnp.dot(h2b, sgu_ref[...].astype(jnp.bfloat16),
                preferred_element_type=jnp.float32)
    g1 = g[:, :_ISH]
    g2 = g[:, _ISH:]
    a = (g1 * jax.nn.sigmoid(g1) * g2).astype(jnp.bfloat16)
    shared_ref[...] = jnp.dot(a, sdn_ref[...].astype(jnp.bfloat16),
                              preferred_element_type=jnp.float32)


def _moe_kernel(h2_ref, we_ref, shared_ref, gu_ref, dn_ref, out_ref):
    e = pl.program_id(0)
    r = pl.program_id(1)
    g = jnp.dot(h2_ref[...], gu_ref[0].astype(jnp.bfloat16),
                preferred_element_type=jnp.float32)
    g1 = g[:, :_IM]
    g2 = g[:, _IM:]
    a = (g1 * jax.nn.sigmoid(g1) * g2).astype(jnp.bfloat16)
    xe = jnp.dot(a, dn_ref[0].astype(jnp.bfloat16),
                 preferred_element_type=jnp.float32)
    idx = jax.lax.broadcasted_iota(jnp.int32, (_BSM, _E), 1)
    w = jnp.sum(we_ref[...] * (idx == e).astype(jnp.float32), axis=1,
                keepdims=True)
    contrib = w * xe
    rows = pl.ds(r * _BSM, _BSM)

    @pl.when(e == 0)
    def _():
        out_ref[rows, :] = shared_ref[...] + contrib

    @pl.when(e != 0)
    def _():
        out_ref[rows, :] = out_ref[rows, :] + contrib


def kernel(hidden_states, residual, w_in_ln, W_qkv, W_o, w_post_ln, gate_w,
           gate_bias, exp_gu, exp_dn, sh_gu, sh_dn, positions):
    T = hidden_states.shape[0]

    head = np.concatenate([np.arange(0, _HD, 2), np.arange(1, _HD, 2)])
    perm = np.concatenate(
        [h0 + head for h0 in range(0, _NQ + _NKVD, _HD)]
        + [np.arange(_NQ + _NKVD, _QKVW)])
    wqkv_p = W_qkv[:, perm]

    inv_freq = 1.0 / (_THETA ** (jnp.arange(0, _HD, 2, dtype=jnp.float32) / _HD))
    freqs = positions.astype(jnp.float32)[:, None] * inv_freq[None, :]
    cos_t = jnp.cos(freqs)
    sin_t = jnp.sin(freqs)

    nR = T // _BS

    res1, q, k, v = pl.pallas_call(
        _pre_kernel,
        grid=(nR,),
        in_specs=[
            pl.BlockSpec((_BS, _H), lambda r: (r, 0)),
            pl.BlockSpec((_BS, _H), lambda r: (r, 0)),
            pl.BlockSpec((1, _H), lambda r: (0, 0)),
            pl.BlockSpec((_H, _QKVW), lambda r: (0, 0)),
            pl.BlockSpec((_BS, _HD // 2), lambda r: (r, 0)),
            pl.BlockSpec((_BS, _HD // 2), lambda r: (r, 0)),
        ],
        out_specs=[
            pl.BlockSpec((_BS, _H), lambda r: (r, 0)),
            pl.BlockSpec((_BS, _NQ), lambda r: (r, 0)),
            pl.BlockSpec((_NKV, _BS, _HD), lambda r: (0, r, 0)),
            pl.BlockSpec((_NKV, _BS, _HD), lambda r: (0, r, 0)),
        ],
        out_shape=[
            jax.ShapeDtypeStruct((T, _H), jnp.float32),
            jax.ShapeDtypeStruct((T, _NQ), jnp.float32),
            jax.ShapeDtypeStruct((_NKV, T, _HD), jnp.float32),
            jax.ShapeDtypeStruct((_NKV, T, _HD), jnp.float32),
        ],
    )(hidden_states, residual, w_in_ln.reshape(1, _H), wqkv_p, cos_t, sin_t)

    ctx = pl.pallas_call(
        _attn_kernel,
        grid=(_NKV, T // _BQ),
        in_specs=[
            pl.BlockSpec((_BQ, 4 * _HD), lambda g, i: (i, g)),
            pl.BlockSpec((1, T, _HD), lambda g, i: (g, 0, 0)),
            pl.BlockSpec((1, T, _HD), lambda g, i: (g, 0, 0)),
        ],
        out_specs=pl.BlockSpec((_BQ, 4 * _HD), lambda g, i: (i, g)),
        out_shape=jax.ShapeDtypeStruct((T, _NQ), jnp.float32),
        scratch_shapes=[pltpu.VMEM((4 * _BQ, T), jnp.float32)],
    )(q, k, v)

    res2, h2, we, shared = pl.pallas_call(
        _post_kernel,
        grid=(nR,),
        in_specs=[
            pl.BlockSpec((_BS, _NQ), lambda r: (r, 0)),
            pl.BlockSpec((_BS, _H), lambda r: (r, 0)),
            pl.BlockSpec((1, _H), lambda r: (0, 0)),
            pl.BlockSpec((_NQ, _H), lambda r: (0, 0)),
            pl.BlockSpec((_H, _E), lambda r: (0, 0)),
            pl.BlockSpec((1, _E), lambda r: (0, 0)),
            pl.BlockSpec((_H, 2 * _ISH), lambda r: (0, 0)),
            pl.BlockSpec((_ISH, _H), lambda r: (0, 0)),
        ],
        out_specs=[
            pl.BlockSpec((_BS, _H), lambda r: (r, 0)),
            pl.BlockSpec((_BS, _H), lambda r: (r, 0)),
            pl.BlockSpec((_BS, _E), lambda r: (r, 0)),
            pl.BlockSpec((_BS, _H), lambda r: (r, 0)),
        ],
        out_shape=[
            jax.ShapeDtypeStruct((T, _H), jnp.float32),
            jax.ShapeDtypeStruct((T, _H), jnp.bfloat16),
            jax.ShapeDtypeStruct((T, _E), jnp.float32),
            jax.ShapeDtypeStruct((T, _H), jnp.float32),
        ],
    )(ctx, res1, w_post_ln.reshape(1, _H), W_o, gate_w.T, gate_bias,
      sh_gu, sh_dn)

    h_out = pl.pallas_call(
        _moe_kernel,
        grid=(_E, T // _BSM),
        in_specs=[
            pl.BlockSpec((_BSM, _H), lambda e, r: (r, 0)),
            pl.BlockSpec((_BSM, _E), lambda e, r: (r, 0)),
            pl.BlockSpec((_BSM, _H), lambda e, r: (r, 0)),
            pl.BlockSpec((1, _H, 2 * _IM), lambda e, r: (e, 0, 0)),
            pl.BlockSpec((1, _IM, _H), lambda e, r: (e, 0, 0)),
        ],
        out_specs=pl.BlockSpec((T, _H), lambda e, r: (0, 0)),
        out_shape=jax.ShapeDtypeStruct((T, _H), jnp.float32),
    )(h2, we, shared, exp_gu, exp_dn)

    return h_out, res2


# speedup vs baseline: 2.3784x; 1.0465x over previous
import numpy as np
import jax
import jax.numpy as jnp
from jax.experimental import pallas as pl
from jax.experimental.pallas import tpu as pltpu

_H = 1024
_NH = 16
_NKV = 4
_HD = 64
_E = 8
_IM = 512
_ISH = 1024
_THETA = 500000.0
_EPS = 1e-06

_BS = 256
_BQ = 512
_BK = 512
_BSM = 512

_NQ = _NH * _HD
_NKVD = _NKV * _HD
_QKVW = _NQ + 2 * _NKVD


def _rope_head(x, cos, sin):
    x1 = x[:, :_HD // 2]
    x2 = x[:, _HD // 2:]
    return jnp.concatenate([x1 * cos - x2 * sin, x1 * sin + x2 * cos], axis=1)


def _pre_kernel(hid_ref, res_ref, wln_ref, wqkv_ref, c_ref, s_ref,
                res1_ref, q_ref, k_ref, v_ref):
    h = hid_ref[...] + res_ref[...]
    res1_ref[...] = h
    var = jnp.mean(h * h, axis=1, keepdims=True)
    ln = h * jax.lax.rsqrt(var + _EPS) * wln_ref[...]
    qkv = jnp.dot(ln, wqkv_ref[...], preferred_element_type=jnp.float32)
    c = c_ref[...]
    s = s_ref[...]
    q_ref[...] = jnp.concatenate(
        [_rope_head(qkv[:, h0:h0 + _HD], c, s)
         for h0 in range(0, _NQ, _HD)], axis=1)
    k_ref[...] = jnp.stack(
        [_rope_head(qkv[:, h0:h0 + _HD], c, s)
         for h0 in range(_NQ, _NQ + _NKVD, _HD)], axis=0)
    v_ref[...] = jnp.stack(
        [qkv[:, h0:h0 + _HD]
         for h0 in range(_NQ + _NKVD, _QKVW, _HD)], axis=0)


def _attn_kernel(q_ref, k_ref, v_ref, o_ref, s_ref):
    i = pl.program_id(1)
    qb = q_ref[...]
    q = jnp.concatenate([qb[:, h0:h0 + _HD]
                         for h0 in range(0, 4 * _HD, _HD)], axis=0) * 0.125
    row = jax.lax.broadcasted_iota(jnp.int32, (4 * _BQ, _BK), 0)
    qpos = i * _BQ + (row & (_BQ - 1))
    col = jax.lax.broadcasted_iota(jnp.int32, (4 * _BQ, _BK), 1)

    def body1(j, m):
        kj = k_ref[0, pl.ds(j * _BK, _BK), :]
        s = jax.lax.dot_general(q, kj, (((1,), (1,)), ((), ())),
                                preferred_element_type=jnp.float32)
        s = jnp.where(qpos >= col + j * _BK, s, -1e30)
        s_ref[:, pl.ds(j * _BK, _BK)] = s
        return jnp.maximum(m, jnp.max(s, axis=1, keepdims=True))

    nkb = ((i + 1) * _BQ + _BK - 1) // _BK
    m = jax.lax.fori_loop(0, nkb, body1,
                          jnp.full((4 * _BQ, 1), -1e30, jnp.float32))

    def body2(j, carry):
        acc, l = carry
        cols = pl.ds(j * _BK, _BK)
        p = jnp.exp(s_ref[:, cols] - m)
        vj = v_ref[0, cols, :]
        l = l + jnp.sum(p, axis=1, keepdims=True)
        acc = acc + jnp.dot(p, vj, preferred_element_type=jnp.float32)
        return acc, l

    acc, l = jax.lax.fori_loop(
        0, nkb, body2,
        (jnp.zeros((4 * _BQ, _HD), jnp.float32),
         jnp.zeros((4 * _BQ, 1), jnp.float32)))
    acc = acc * (1.0 / l)
    o_ref[...] = jnp.concatenate([acc[h0 * _BQ:(h0 + 1) * _BQ, :]
                                  for h0 in range(4)], axis=1)


def _post_kernel(ctx_ref, res1_ref, wln_ref, wo_ref, gwt_ref, gb_ref,
                 sgu_ref, sdn_ref, res2_ref, h2_ref, we_ref, shared_ref):
    att = jnp.dot(ctx_ref[...], wo_ref[...], preferred_element_type=jnp.float32)
    r2 = att + res1_ref[...]
    res2_ref[...] = r2
    var = jnp.mean(r2 * r2, axis=1, keepdims=True)
    h2 = r2 * jax.lax.rsqrt(var + _EPS) * wln_ref[...]
    h2b = h2.astype(jnp.bfloat16)
    h2_ref[...] = h2b
    logits = jnp.dot(h2, gwt_ref[...], preferred_element_type=jnp.float32)
    mx = jnp.max(logits, axis=1, keepdims=True)
    ex = jnp.exp(logits - mx)
    probs = ex / jnp.sum(ex, axis=1, keepdims=True)
    b = probs + gb_ref[...]
    idx = jax.lax.broadcasted_iota(jnp.int32, (_BS, _E), 1)
    m1 = jnp.max(b, axis=1, keepdims=True)
    a1 = jnp.min(jnp.where(b == m1, idx, _E), axis=1, keepdims=True)
    oh1 = idx == a1
    b2 = jnp.where(oh1, -1e30, b)
    m2 = jnp.max(b2, axis=1, keepdims=True)
    a2 = jnp.min(jnp.where(b2 == m2, idx, _E), axis=1, keepdims=True)
    sel = oh1 | (idx == a2)
    w = jnp.where(sel, probs, 0.0)
    we_ref[...] = w / jnp.sum(w, axis=1, keepdims=True)
    g = jnp.dot(h2b, sgu_ref[...].astype(jnp.bfloat16),
                preferred_element_type=jnp.float32)
    g1 = g[:, :_ISH]
    g2 = g[:, _ISH:]
    a = (g1 * jax.nn.sigmoid(g1) * g2).astype(jnp.bfloat16)
    shared_ref[...] = jnp.dot(a, sdn_ref[...].astype(jnp.bfloat16),
                              preferred_element_type=jnp.float32)


def _moe_kernel(h2_ref, we_ref, shared_ref, gu_ref, dn_ref, out_ref):
    e = pl.program_id(0)
    r = pl.program_id(1)
    g = jnp.dot(h2_ref[...], gu_ref[0].astype(jnp.bfloat16),
                preferred_element_type=jnp.float32)
    g1 = g[:, :_IM]
    g2 = g[:, _IM:]
    a = (g1 * jax.nn.sigmoid(g1) * g2).astype(jnp.bfloat16)
    xe = jnp.dot(a, dn_ref[0].astype(jnp.bfloat16),
                 preferred_element_type=jnp.float32)
    idx = jax.lax.broadcasted_iota(jnp.int32, (_BSM, _E), 1)
    w = jnp.sum(we_ref[...] * (idx == e).astype(jnp.float32), axis=1,
                keepdims=True)
    contrib = w * xe
    rows = pl.ds(r * _BSM, _BSM)

    @pl.when(e == 0)
    def _():
        out_ref[rows, :] = shared_ref[...] + contrib

    @pl.when(e != 0)
    def _():
        out_ref[rows, :] = out_ref[rows, :] + contrib


def kernel(hidden_states, residual, w_in_ln, W_qkv, W_o, w_post_ln, gate_w,
           gate_bias, exp_gu, exp_dn, sh_gu, sh_dn, positions):
    T = hidden_states.shape[0]

    head = np.concatenate([np.arange(0, _HD, 2), np.arange(1, _HD, 2)])
    perm = np.concatenate(
        [h0 + head for h0 in range(0, _NQ + _NKVD, _HD)]
        + [np.arange(_NQ + _NKVD, _QKVW)])
    wqkv_p = W_qkv[:, perm]

    inv_freq = 1.0 / (_THETA ** (jnp.arange(0, _HD, 2, dtype=jnp.float32) / _HD))
    freqs = positions.astype(jnp.float32)[:, None] * inv_freq[None, :]
    cos_t = jnp.cos(freqs)
    sin_t = jnp.sin(freqs)

    nR = T // _BS

    res1, q, k, v = pl.pallas_call(
        _pre_kernel,
        grid=(nR,),
        in_specs=[
            pl.BlockSpec((_BS, _H), lambda r: (r, 0)),
            pl.BlockSpec((_BS, _H), lambda r: (r, 0)),
            pl.BlockSpec((1, _H), lambda r: (0, 0)),
            pl.BlockSpec((_H, _QKVW), lambda r: (0, 0)),
            pl.BlockSpec((_BS, _HD // 2), lambda r: (r, 0)),
            pl.BlockSpec((_BS, _HD // 2), lambda r: (r, 0)),
        ],
        out_specs=[
            pl.BlockSpec((_BS, _H), lambda r: (r, 0)),
            pl.BlockSpec((_BS, _NQ), lambda r: (r, 0)),
            pl.BlockSpec((_NKV, _BS, _HD), lambda r: (0, r, 0)),
            pl.BlockSpec((_NKV, _BS, _HD), lambda r: (0, r, 0)),
        ],
        out_shape=[
            jax.ShapeDtypeStruct((T, _H), jnp.float32),
            jax.ShapeDtypeStruct((T, _NQ), jnp.float32),
            jax.ShapeDtypeStruct((_NKV, T, _HD), jnp.float32),
            jax.ShapeDtypeStruct((_NKV, T, _HD), jnp.float32),
        ],
    )(hidden_states, residual, w_in_ln.reshape(1, _H), wqkv_p, cos_t, sin_t)

    ctx = pl.pallas_call(
        _attn_kernel,
        grid=(_NKV, T // _BQ),
        in_specs=[
            pl.BlockSpec((_BQ, 4 * _HD), lambda g, i: (i, g)),
            pl.BlockSpec((1, T, _HD), lambda g, i: (g, 0, 0)),
            pl.BlockSpec((1, T, _HD), lambda g, i: (g, 0, 0)),
        ],
        out_specs=pl.BlockSpec((_BQ, 4 * _HD), lambda g, i: (i, g)),
        out_shape=jax.ShapeDtypeStruct((T, _NQ), jnp.float32),
        scratch_shapes=[pltpu.VMEM((4 * _BQ, T), jnp.float32)],
    )(q, k, v)

    res2, h2, we, shared = pl.pallas_call(
        _post_kernel,
        grid=(nR,),
        in_specs=[
            pl.BlockSpec((_BS, _NQ), lambda r: (r, 0)),
            pl.BlockSpec((_BS, _H), lambda r: (r, 0)),
            pl.BlockSpec((1, _H), lambda r: (0, 0)),
            pl.BlockSpec((_NQ, _H), lambda r: (0, 0)),
            pl.BlockSpec((_H, _E), lambda r: (0, 0)),
            pl.BlockSpec((1, _E), lambda r: (0, 0)),
            pl.BlockSpec((_H, 2 * _ISH), lambda r: (0, 0)),
            pl.BlockSpec((_ISH, _H), lambda r: (0, 0)),
        ],
        out_specs=[
            pl.BlockSpec((_BS, _H), lambda r: (r, 0)),
            pl.BlockSpec((_BS, _H), lambda r: (r, 0)),
            pl.BlockSpec((_BS, _E), lambda r: (r, 0)),
            pl.BlockSpec((_BS, _H), lambda r: (r, 0)),
        ],
        out_shape=[
            jax.ShapeDtypeStruct((T, _H), jnp.float32),
            jax.ShapeDtypeStruct((T, _H), jnp.bfloat16),
            jax.ShapeDtypeStruct((T, _E), jnp.float32),
            jax.ShapeDtypeStruct((T, _H), jnp.float32),
        ],
    )(ctx, res1, w_post_ln.reshape(1, _H), W_o, gate_w.T, gate_bias,
      sh_gu, sh_dn)

    h_out = pl.pallas_call(
        _moe_kernel,
        grid=(_E, T // _BSM),
        in_specs=[
            pl.BlockSpec((_BSM, _H), lambda e, r: (r, 0)),
            pl.BlockSpec((_BSM, _E), lambda e, r: (r, 0)),
            pl.BlockSpec((_BSM, _H), lambda e, r: (r, 0)),
            pl.BlockSpec((1, _H, 2 * _IM), lambda e, r: (e, 0, 0)),
            pl.BlockSpec((1, _IM, _H), lambda e, r: (e, 0, 0)),
        ],
        out_specs=pl.BlockSpec((T, _H), lambda e, r: (0, 0)),
        out_shape=jax.ShapeDtypeStruct((T, _H), jnp.float32),
    )(h2, we, shared, exp_gu, exp_dn)

    return h_out, res2
